# baseline (speedup 1.0000x reference)
import functools

import jax
import jax.numpy as jnp
import numpy as np
from jax.experimental import pallas as pl

CLUSTERS = 128
ITERATIONS = 10
BITS = 32


def _attn_body(q_ref, k_ref, v_ref, kadd_ref, planes_ref, out_ref):
    L, E = q_ref.shape[1], q_ref.shape[2]
    C = CLUSTERS
    q = q_ref[0]
    w = planes_ref[:, :E]
    b = planes_ref[:, E]

    proj = jnp.dot(q, w.T, preferred_element_type=jnp.float32) + b[None, :]
    hb = (proj > 0.0).astype(jnp.float32)
    hb_aug16 = jnp.concatenate(
        [hb, jnp.ones((L, 1), jnp.float32)], axis=1).astype(jnp.bfloat16)
    hb16 = hb.astype(jnp.bfloat16)

    iota_cl_c = jax.lax.broadcasted_iota(jnp.int32, (C, L), 0)
    iota_cl_l = jax.lax.broadcasted_iota(jnp.int32, (C, L), 1)
    sel0 = (iota_cl_l == (L // C) * iota_cl_c).astype(jnp.float32)
    cb0 = jax.lax.dot_general(
        sel0, hb, (((1,), (0,)), ((), ())),
        preferred_element_type=jnp.float32)

    iota_c = jax.lax.broadcasted_iota(
        jnp.int32, (C, 1), 0).astype(jnp.float32)

    def lloyd(_, carry):
        cb, _, _ = carry
        cpop = jnp.sum(cb, axis=-1, keepdims=True)
        dot = jax.lax.dot_general(
            cb.astype(jnp.bfloat16), hb16, (((1,), (1,)), ((), ())),
            preferred_element_type=jnp.float32)
        key = (dot * -256.0) + (cpop * 128.0 + iota_c)
        kmin = jnp.min(key, axis=0, keepdims=True)
        onehot_t = (key == kmin).astype(jnp.float32)
        bitsum_aug = jax.lax.dot_general(
            onehot_t.astype(jnp.bfloat16), hb_aug16, (((1,), (0,)), ((), ())),
            preferred_element_type=jnp.float32)
        bitsum = bitsum_aug[:, :BITS]
        counts = bitsum_aug[:, BITS:]
        newcb = (bitsum / jnp.maximum(counts, 1.0) > 0.5).astype(jnp.float32)
        cb = jnp.where(counts > 0.0, newcb, cb)
        return cb, onehot_t, counts

    init = (cb0, jnp.zeros((C, L), jnp.float32), jnp.zeros((C, 1), jnp.float32))
    _, onehot_t, counts = jax.lax.fori_loop(0, ITERATIONS, lloyd, init)

    factors = 1.0 / jnp.maximum(counts, 1.0)
    q_grouped = jax.lax.dot_general(
        onehot_t, q, (((1,), (0,)), ((), ())),
        preferred_element_type=jnp.float32)
    q_grouped = q_grouped * factors

    k = k_ref[0]
    qk = jax.lax.dot_general(
        q_grouped, k, (((1,), (1,)), ((), ())),
        preferred_element_type=jnp.float32)
    qk = qk + kadd_ref[0, 0][None, :]
    temp = 1.0 / np.sqrt(E).astype(np.float32)
    p = jnp.exp(temp * qk)
    z = jnp.sum(p, axis=-1, keepdims=True)
    v_grouped = jnp.dot(p, v_ref[0], preferred_element_type=jnp.float32)
    v_grouped = v_grouped / z

    out_ref[0] = jax.lax.dot_general(
        onehot_t, v_grouped, (((0,), (0,)), ((), ())),
        preferred_element_type=jnp.float32)


@jax.jit
def kernel(queries, keys, values, key_lengths_additive, planes):
    N, L, H, E = queries.shape
    NH = N * H
    q = jnp.transpose(queries, (0, 2, 1, 3)).reshape(NH, L, E)
    k = jnp.transpose(keys, (0, 2, 1, 3)).reshape(NH, L, E)
    v = jnp.transpose(values, (0, 2, 1, 3)).reshape(NH, L, E)
    kadd = key_lengths_additive.reshape(N, 1, L)

    out = pl.pallas_call(
        _attn_body,
        grid=(NH,),
        in_specs=[
            pl.BlockSpec((1, L, E), lambda i: (i, 0, 0)),
            pl.BlockSpec((1, L, E), lambda i: (i, 0, 0)),
            pl.BlockSpec((1, L, E), lambda i: (i, 0, 0)),
            pl.BlockSpec((1, 1, L), lambda i: (i // H, 0, 0)),
            pl.BlockSpec((BITS, E + 1), lambda i: (0, 0)),
        ],
        out_specs=pl.BlockSpec((1, L, E), lambda i: (i, 0, 0)),
        out_shape=jax.ShapeDtypeStruct((NH, L, E), jnp.float32),
    )(q, k, v, kadd, planes)

    return jnp.transpose(out.reshape(N, H, L, E), (0, 2, 1, 3))


# speedup vs baseline: 1.0391x; 1.0391x over previous
import functools

import jax
import jax.numpy as jnp
import numpy as np
from jax.experimental import pallas as pl

CLUSTERS = 128
ITERATIONS = 10
BITS = 32


def _attn_body(q_ref, k_ref, v_ref, kadd_ref, planes_ref, out_ref):
    L, E = q_ref.shape[1], q_ref.shape[2]
    C = CLUSTERS
    q = q_ref[0]
    w = planes_ref[:, :E]
    b = planes_ref[:, E]

    proj = jnp.dot(q, w.T, preferred_element_type=jnp.float32) + b[None, :]
    hb = (proj > 0.0).astype(jnp.float32)
    hb_aug = jnp.concatenate([hb, jnp.ones((L, 1), jnp.float32)], axis=1)

    iota_cl_c = jax.lax.broadcasted_iota(jnp.int32, (C, L), 0)
    iota_cl_l = jax.lax.broadcasted_iota(jnp.int32, (C, L), 1)
    sel0 = (iota_cl_l == (L // C) * iota_cl_c).astype(jnp.float32)
    cb0 = jax.lax.dot_general(
        sel0, hb, (((1,), (0,)), ((), ())),
        preferred_element_type=jnp.float32)

    iota_c = jax.lax.broadcasted_iota(
        jnp.int32, (C, 1), 0).astype(jnp.float32)

    def lloyd(_, carry):
        cb, _, _ = carry
        cpop = jnp.sum(cb, axis=-1, keepdims=True)
        dot = jax.lax.dot_general(
            cb, hb, (((1,), (1,)), ((), ())),
            preferred_element_type=jnp.float32)
        key = (dot * -256.0) + (cpop * 128.0 + iota_c)
        kmin = jnp.min(key, axis=0, keepdims=True)
        onehot_t = (key == kmin).astype(jnp.float32)
        bitsum_aug = jax.lax.dot_general(
            onehot_t, hb_aug, (((1,), (0,)), ((), ())),
            preferred_element_type=jnp.float32)
        bitsum = bitsum_aug[:, :BITS]
        counts = bitsum_aug[:, BITS:]
        newcb = (bitsum / jnp.maximum(counts, 1.0) > 0.5).astype(jnp.float32)
        cb = jnp.where(counts > 0.0, newcb, cb)
        return cb, onehot_t, counts

    init = (cb0, jnp.zeros((C, L), jnp.float32), jnp.zeros((C, 1), jnp.float32))
    _, onehot_t, counts = jax.lax.fori_loop(0, ITERATIONS, lloyd, init)

    factors = 1.0 / jnp.maximum(counts, 1.0)
    q_grouped = jax.lax.dot_general(
        onehot_t, q, (((1,), (0,)), ((), ())),
        preferred_element_type=jnp.float32)
    q_grouped = q_grouped * factors

    k = k_ref[0]
    qk = jax.lax.dot_general(
        q_grouped, k, (((1,), (1,)), ((), ())),
        preferred_element_type=jnp.float32)
    qk = qk + kadd_ref[0, 0][None, :]
    temp = 1.0 / np.sqrt(E).astype(np.float32)
    p = jnp.exp(temp * qk)
    z = jnp.sum(p, axis=-1, keepdims=True)
    v_grouped = jnp.dot(p, v_ref[0], preferred_element_type=jnp.float32)
    v_grouped = v_grouped / z

    out_ref[0] = jax.lax.dot_general(
        onehot_t, v_grouped, (((0,), (0,)), ((), ())),
        preferred_element_type=jnp.float32)


@jax.jit
def kernel(queries, keys, values, key_lengths_additive, planes):
    N, L, H, E = queries.shape
    NH = N * H
    q = jnp.transpose(queries, (0, 2, 1, 3)).reshape(NH, L, E)
    k = jnp.transpose(keys, (0, 2, 1, 3)).reshape(NH, L, E)
    v = jnp.transpose(values, (0, 2, 1, 3)).reshape(NH, L, E)
    kadd = key_lengths_additive.reshape(N, 1, L)

    out = pl.pallas_call(
        _attn_body,
        grid=(NH,),
        in_specs=[
            pl.BlockSpec((1, L, E), lambda i: (i, 0, 0)),
            pl.BlockSpec((1, L, E), lambda i: (i, 0, 0)),
            pl.BlockSpec((1, L, E), lambda i: (i, 0, 0)),
            pl.BlockSpec((1, 1, L), lambda i: (i // H, 0, 0)),
            pl.BlockSpec((BITS, E + 1), lambda i: (0, 0)),
        ],
        out_specs=pl.BlockSpec((1, L, E), lambda i: (i, 0, 0)),
        out_shape=jax.ShapeDtypeStruct((NH, L, E), jnp.float32),
    )(q, k, v, kadd, planes)

    return jnp.transpose(out.reshape(N, H, L, E), (0, 2, 1, 3))
